# baseline (speedup 1.0000x reference)
import functools
import jax
import jax.numpy as jnp
from jax import lax
from jax.experimental import pallas as pl
from jax.experimental.pallas import tpu as pltpu
from jax.experimental.pallas import tpu_sc as plsc

_NCENT = 512
_NCB = 32
_SUB = 64
_BT = 256
_DIN = _NCB * _SUB
_CHUNK = 128


def _codes_body(x_ref, cent_ref, caug_ref, negmse_ref, codes_ref, flat_ref):
    x = x_ref[...]
    negs, codes, flats = [], [], []
    for c in range(_NCB):
        xc = x[:, c * _SUB:(c + 1) * _SUB]
        cc = cent_ref[c]
        resp = lax.dot_general(xc, cc, (((1,), (1,)), ((), ())))
        m = jnp.max(resp, axis=-1, keepdims=True)
        eqf = (resp >= m).astype(jnp.float32)
        g = lax.dot_general(eqf, caug_ref[c], (((1,), (0,)), ((), ())),
                            preferred_element_type=jnp.float32)
        code = g[:, 0:1] * 16.0 + g[:, 1:2]
        negs.append(-m)
        codes.append(code.astype(jnp.int32))
        flats.append(code.astype(jnp.int32) + c * _NCENT)
    negmse_ref[...] = jnp.concatenate(negs, axis=1)
    codes_ref[...] = jnp.concatenate(codes, axis=1)
    flat_ref[...] = jnp.concatenate(flats, axis=1)


def _codes_call(x2, centroids, caug):
    B = x2.shape[0]
    return pl.pallas_call(
        _codes_body,
        grid=(B // _BT,),
        in_specs=[
            pl.BlockSpec((_BT, _DIN), lambda i: (i, 0)),
            pl.BlockSpec((_NCB, _NCENT, _SUB), lambda i: (0, 0, 0)),
            pl.BlockSpec((_NCB, _NCENT, 2), lambda i: (0, 0, 0)),
        ],
        out_specs=(
            pl.BlockSpec((_BT, _NCB), lambda i: (i, 0)),
            pl.BlockSpec((_BT, _NCB), lambda i: (i, 0)),
            pl.BlockSpec((_BT, _NCB), lambda i: (i, 0)),
        ),
        out_shape=(
            jax.ShapeDtypeStruct((B, _NCB), jnp.float32),
            jax.ShapeDtypeStruct((B, _NCB), jnp.int32),
            jax.ShapeDtypeStruct((B, _NCB), jnp.int32),
        ),
        compiler_params=pltpu.CompilerParams(
            dimension_semantics=("arbitrary",),
        ),
    )(x2, centroids, caug)


def _sc_gather(table, flat_idx):
    nrows, d = flat_idx.shape[0], table.shape[1]
    info = plsc.get_sparse_core_info()
    nw = info.num_cores * info.num_subcores
    rows_per_w = nrows // nw
    nchunk = rows_per_w // _CHUNK
    idx3 = flat_idx.reshape(nw, nchunk, _CHUNK)
    mesh = plsc.VectorSubcoreMesh(core_axis_name="c", subcore_axis_name="s")

    @functools.partial(
        pl.kernel,
        out_type=jax.ShapeDtypeStruct((nrows, d), table.dtype),
        mesh=mesh,
        compiler_params=pltpu.CompilerParams(use_tc_tiling_on_sc=False),
        scratch_types=[
            pltpu.VMEM((nchunk, _CHUNK), jnp.int32),
            pltpu.VMEM((_CHUNK, d), table.dtype),
            pltpu.VMEM((_CHUNK, d), table.dtype),
            pltpu.SemaphoreType.DMA,
            pltpu.SemaphoreType.DMA,
        ],
    )
    def gather_k(table_hbm, idx_hbm, out_hbm, idx_v, buf0, buf1, sem0, sem1):
        wid = lax.axis_index("s") * info.num_cores + lax.axis_index("c")
        base = wid * rows_per_w
        pltpu.sync_copy(idx_hbm.at[wid], idx_v)
        bufs = (buf0, buf1)
        sems = (sem0, sem1)

        def body(k0, carry):
            cps = []
            for b in range(2):
                k = k0 * 2 + b
                cps.append(pltpu.async_copy(
                    table_hbm.at[idx_v.at[k]], bufs[b], sems[b]))
            for b in range(2):
                k = k0 * 2 + b
                cps[b].wait()
                pltpu.sync_copy(
                    bufs[b], out_hbm.at[pl.ds(base + k * _CHUNK, _CHUNK)])
            return carry

        lax.fori_loop(0, nchunk // 2, body, 0)

    return gather_k(table, idx3)


def _matmul_body(x_ref, w_ref, o_ref):
    o_ref[...] = lax.dot_general(x_ref[...], w_ref[...],
                                 (((1,), (0,)), ((), ())),
                                 preferred_element_type=jnp.float32)


def _matmul_call(x, w_bf):
    B, out_dim = x.shape[0], w_bf.shape[1]
    return pl.pallas_call(
        _matmul_body,
        grid=(B // _BT,),
        in_specs=[
            pl.BlockSpec((_BT, _DIN), lambda i: (i, 0)),
            pl.BlockSpec((_DIN, out_dim), lambda i: (0, 0)),
        ],
        out_specs=pl.BlockSpec((_BT, out_dim), lambda i: (i, 0)),
        out_shape=jax.ShapeDtypeStruct((B, out_dim), jnp.float32),
        compiler_params=pltpu.CompilerParams(
            dimension_semantics=("arbitrary",),
        ),
    )(x, w_bf)


def kernel(inputs, centroids, W):
    B = inputs.shape[0]
    x2 = inputs.reshape(B, _DIN)
    k = jnp.arange(_NCENT, dtype=jnp.int32)
    idx_cols = jnp.stack([(k // 16).astype(jnp.float32),
                          (k % 16).astype(jnp.float32)], axis=1)
    caug = jnp.broadcast_to(idx_cols[None], (_NCB, _NCENT, 2))
    negmse, codes, flat = _codes_call(x2, centroids, caug)
    table = centroids.reshape(_NCB * _NCENT, _SUB)
    rows = _sc_gather(table, flat.reshape(-1))
    prod = _matmul_call(rows.reshape(B, _DIN), W)
    return (prod, negmse, codes)


# speedup vs baseline: 2.4228x; 2.4228x over previous
import jax
import jax.numpy as jnp
from jax import lax
from jax.experimental import pallas as pl
from jax.experimental.pallas import tpu as pltpu

_NCENT = 512
_NCB = 32
_SUB = 64
_BT = 256
_DIN = _NCB * _SUB
_AUG = _SUB + 2


def _fused_body(x_ref, caug_hbm, w_hbm, prod_ref, negmse_ref, codes_ref,
                outs_ref, caug_ref, w_ref, sem0, sem1):
    @pl.when(pl.program_id(0) == 0)
    def _load_consts():
        cp0 = pltpu.make_async_copy(caug_hbm, caug_ref, sem0)
        cp1 = pltpu.make_async_copy(w_hbm, w_ref, sem1)
        cp0.start()
        cp1.start()
        cp0.wait()
        cp1.wait()

    x = x_ref[...]
    negs = []
    codes = []
    for c in range(_NCB):
        xc = x[:, c * _SUB:(c + 1) * _SUB]
        cc = caug_ref[c][:, :_SUB]
        resp = lax.dot_general(xc, cc, (((1,), (1,)), ((), ())))
        m = jnp.max(resp, axis=-1, keepdims=True)
        eqf = (resp >= m).astype(jnp.float32)
        g = lax.dot_general(eqf, caug_ref[c], (((1,), (0,)), ((), ())),
                            preferred_element_type=jnp.float32)
        outs_ref[:, c * _SUB:(c + 1) * _SUB] = g[:, :_SUB]
        code = (g[:, _SUB:_SUB + 1] * 16.0 + g[:, _SUB + 1:_SUB + 2])
        codes.append(code.astype(jnp.int32))
        negs.append(-m)
    negmse_ref[...] = jnp.concatenate(negs, axis=1)
    codes_ref[...] = jnp.concatenate(codes, axis=1)
    prod_ref[...] = lax.dot_general(outs_ref[...], w_ref[...],
                                    (((1,), (0,)), ((), ())),
                                    preferred_element_type=jnp.float32)


def kernel(inputs, centroids, W):
    B = inputs.shape[0]
    out_dim = W.shape[1]
    x2 = inputs.reshape(B, _DIN)
    k = jnp.arange(_NCENT, dtype=jnp.int32)
    khi = (k // 16).astype(jnp.float32)
    klo = (k % 16).astype(jnp.float32)
    idx_cols = jnp.stack([khi, klo], axis=1)
    caug = jnp.concatenate(
        [centroids,
         jnp.broadcast_to(idx_cols[None], (_NCB, _NCENT, 2))], axis=2)
    grid = (B // _BT,)
    prod, negmse, codes = pl.pallas_call(
        _fused_body,
        grid=grid,
        in_specs=[
            pl.BlockSpec((_BT, _DIN), lambda i: (i, 0)),
            pl.BlockSpec(memory_space=pl.ANY),
            pl.BlockSpec(memory_space=pl.ANY),
        ],
        out_specs=(
            pl.BlockSpec((_BT, out_dim), lambda i: (i, 0)),
            pl.BlockSpec((_BT, _NCB), lambda i: (i, 0)),
            pl.BlockSpec((_BT, _NCB), lambda i: (i, 0)),
        ),
        out_shape=(
            jax.ShapeDtypeStruct((B, out_dim), jnp.float32),
            jax.ShapeDtypeStruct((B, _NCB), jnp.float32),
            jax.ShapeDtypeStruct((B, _NCB), jnp.int32),
        ),
        scratch_shapes=[
            pltpu.VMEM((_BT, _DIN), jnp.float32),
            pltpu.VMEM((_NCB, _NCENT, _AUG), jnp.float32),
            pltpu.VMEM((_DIN, out_dim), jnp.float32),
            pltpu.SemaphoreType.DMA,
            pltpu.SemaphoreType.DMA,
        ],
        compiler_params=pltpu.CompilerParams(
            dimension_semantics=("arbitrary",),
        ),
    )(x2, caug, W)
    return (prod, negmse, codes)
